# Initial kernel scaffold; baseline (speedup 1.0000x reference)
import jax
import jax.numpy as jnp
from jax.experimental import pallas as pl


def kernel(start, mid, end, durations, render_class, max_frames):
    raise NotImplementedError("write your pallas kernel here")


# speedup vs baseline: 41.3041x; 41.3041x over previous
import functools

import jax
import jax.numpy as jnp
from jax import lax
from jax.experimental import pallas as pl
from jax.experimental.pallas import tpu as pltpu
from jax.experimental.pallas import tpu_sc as plsc

B, N, FDIM, MAXF = 8, 2048, 256, 8192
L = 16
NC, NS = 2, 16
NW = NC * NS
TPB = NW // B
FPT = MAXF // TPB
BLK = 64
NBLK = FPT // BLK
CHUNKS = FDIM // L

_mesh = plsc.VectorSubcoreMesh(core_axis_name="c", subcore_axis_name="s")

_f32 = jnp.float32
_i32 = jnp.int32


def _weights(fv, j, d, cls, total_eff):
    jf = j.astype(_f32)
    half = d // 2
    rem = d - half
    one = jnp.ones((L,), _f32)
    zero = jnp.zeros((L,), _f32)
    t1 = jf / jnp.maximum(half - 1, 1).astype(_f32)
    t2 = (jf - half.astype(_f32)) / jnp.maximum(rem - 1, 1).astype(_f32)
    lt = j < half
    ws_lin = jnp.where(lt, 1.0 - t1, zero)
    wm_lin = jnp.where(lt, t1, 1.0 - t2)
    we_lin = jnp.where(lt, zero, t2)
    j0 = j == 0
    jl = j == d - 1
    ws_pl = jnp.where(j0, one, zero)
    we_pl = jnp.where(jl & ~j0, one, zero)
    wm_pl = jnp.where(~j0 & ~jl, one, zero)
    ispl = cls == 0
    ws4 = jnp.where(ispl, ws_pl, ws_lin)
    wm4 = jnp.where(ispl, wm_pl, wm_lin)
    we4 = jnp.where(ispl, we_pl, we_lin)
    f1 = lambda bcond: jnp.where(bcond, one, zero)
    d1, d2, d3 = d == 1, d == 2, d == 3
    ws = jnp.where(d1, zero, jnp.where(d2, f1(j0), jnp.where(d3, f1(j0), ws4)))
    wm = jnp.where(d1, one, jnp.where(d2, zero, jnp.where(d3, f1(j == 1), wm4)))
    we = jnp.where(d1, zero, jnp.where(d2, f1(~j0), jnp.where(d3, f1(j >= 2), we4)))
    valid = fv < total_eff
    ws = jnp.where(valid, ws, zero)
    wm = jnp.where(valid, wm, zero)
    we = jnp.where(valid, we, zero)
    return ws, wm, we, valid


@functools.partial(
    pl.kernel,
    mesh=_mesh,
    compiler_params=pltpu.CompilerParams(needs_layout_passes=False),
    out_type=(
        jax.ShapeDtypeStruct((B * MAXF, FDIM), _f32),
        jax.ShapeDtypeStruct((B * MAXF,), _i32),
    ),
    scratch_types=[
        pltpu.VMEM((N,), _i32),
        pltpu.VMEM((N,), _i32),
        pltpu.VMEM((N,), _i32),
        pltpu.VMEM((L,), _i32),
        pltpu.VMEM((BLK,), _i32),
        pltpu.VMEM((BLK,), _f32),
        pltpu.VMEM((BLK,), _f32),
        pltpu.VMEM((BLK,), _f32),
        pltpu.VMEM((BLK, FDIM), _f32),
        pltpu.VMEM((BLK, FDIM), _f32),
        pltpu.VMEM((BLK, FDIM), _f32),
        pltpu.VMEM((BLK, FDIM), _f32),
        pltpu.VMEM((FPT,), _i32),
        pltpu.SemaphoreType.DMA,
        pltpu.SemaphoreType.DMA,
        pltpu.SemaphoreType.DMA,
    ],
)
def _render(s_hbm, m_hbm, e_hbm, dur_hbm, cls_hbm, mf_hbm,
            out_hbm, mask_hbm,
            dur_v, csum_v, cls_v, mf_v, idx_v, ws_v, wm_v, we_v,
            s_rows, m_rows, e_rows, out_v, mask_v,
            sem_s, sem_m, sem_e):
    cid = lax.axis_index("c")
    sid = lax.axis_index("s")
    wid = sid * NC + cid
    bidx = wid // TPB
    q = wid % TPB
    f0 = q * FPT

    lane = lax.broadcasted_iota(_i32, (L,), 0)

    pltpu.sync_copy(dur_hbm.at[pl.ds(bidx * N, N)], dur_v)
    pltpu.sync_copy(cls_hbm.at[pl.ds(bidx * N, N)], cls_v)
    pltpu.sync_copy(mf_hbm, mf_v)

    def cs_body(i, carry_v):
        base = i * L
        dv = jnp.maximum(dur_v[pl.ds(base, L)], 0)
        csum_v[pl.ds(base, L)] = dv
        for sh in (1, 2, 4, 8):
            cur = csum_v[pl.ds(base, L)]
            prev = plsc.load_gather(csum_v, [base + jnp.maximum(lane - sh, 0)])
            csum_v[pl.ds(base, L)] = cur + jnp.where(lane >= sh, prev, 0)
        cv = csum_v[pl.ds(base, L)] + carry_v
        csum_v[pl.ds(base, L)] = cv
        return plsc.load_gather(csum_v, [jnp.zeros((L,), _i32) + (base + L - 1)])

    lax.fori_loop(0, N // L, cs_body, jnp.zeros((L,), _i32))

    total_v = plsc.load_gather(csum_v, [jnp.zeros((L,), _i32) + (N - 1)])
    mf_splat = mf_v[...]
    total_eff = jnp.minimum(total_v, mf_splat)

    def blk_body(g, _):
        base = f0 + g * BLK
        for k in range(BLK // L):
            fv = base + k * L + lane
            pos = jnp.zeros((L,), _i32)
            bit = N // 2
            while bit >= 1:
                val = plsc.load_gather(csum_v, [pos + (bit - 1)])
                pos = pos + jnp.where(val <= fv, bit, 0)
                bit //= 2
            ph = pos
            offm = plsc.load_gather(csum_v, [jnp.maximum(ph - 1, 0)])
            off = jnp.where(ph >= 1, offm, 0)
            dsum = plsc.load_gather(csum_v, [ph])
            d = dsum - off
            j = fv - off
            cls = plsc.load_gather(cls_v, [ph])
            ws, wm, we, valid = _weights(fv, j, d, cls, total_eff)
            idx_v[pl.ds(k * L, L)] = bidx * N + ph
            ws_v[pl.ds(k * L, L)] = ws
            wm_v[pl.ds(k * L, L)] = wm
            we_v[pl.ds(k * L, L)] = we
            mask_v[pl.ds(g * BLK + k * L, L)] = valid.astype(_i32)

        cp_s = pltpu.async_copy(s_hbm.at[idx_v], s_rows, sem_s)
        cp_m = pltpu.async_copy(m_hbm.at[idx_v], m_rows, sem_m)
        cp_e = pltpu.async_copy(e_hbm.at[idx_v], e_rows, sem_e)
        cp_s.wait()
        cp_m.wait()
        cp_e.wait()

        def row_body(r, carry):
            rsplat = jnp.zeros((L,), _i32) + r
            wsr = plsc.load_gather(ws_v, [rsplat])
            wmr = plsc.load_gather(wm_v, [rsplat])
            wer = plsc.load_gather(we_v, [rsplat])
            for kk in range(CHUNKS):
                sv = s_rows[r, pl.ds(kk * L, L)]
                mv = m_rows[r, pl.ds(kk * L, L)]
                ev = e_rows[r, pl.ds(kk * L, L)]
                out_v[r, pl.ds(kk * L, L)] = wsr * sv + wmr * mv + wer * ev
            return carry

        lax.fori_loop(0, BLK, row_body, 0)

        pltpu.sync_copy(out_v, out_hbm.at[pl.ds(bidx * MAXF + base, BLK)])
        return _

    lax.fori_loop(0, NBLK, blk_body, 0)
    pltpu.sync_copy(mask_v, mask_hbm.at[pl.ds(bidx * MAXF + f0, FPT)])


def kernel(start, mid, end, durations, render_class, max_frames):
    s = start.reshape(B * N, FDIM)
    m = mid.reshape(B * N, FDIM)
    e = end.reshape(B * N, FDIM)
    dur = durations.reshape(B * N)
    cls = render_class.reshape(B * N)
    mf = jnp.full((L,), max_frames, _i32)
    out_flat, mask_flat = _render(s, m, e, dur, cls, mf)
    out = out_flat.reshape(B, MAXF, FDIM)
    mask = mask_flat.reshape(B, MAXF) != 0
    return out, mask


# speedup vs baseline: 47.7652x; 1.1564x over previous
import functools

import jax
import jax.numpy as jnp
from jax import lax
from jax.experimental import pallas as pl
from jax.experimental.pallas import tpu as pltpu
from jax.experimental.pallas import tpu_sc as plsc

B, N, FDIM, MAXF = 8, 2048, 256, 8192
L = 16
NC, NS = 2, 16
NW = NC * NS
TPB = NW // B
FPT = MAXF // TPB
BLK = 32
NBLK = FPT // BLK
CHUNKS = FDIM // L

_mesh = plsc.VectorSubcoreMesh(core_axis_name="c", subcore_axis_name="s")

_f32 = jnp.float32
_i32 = jnp.int32


def _weights(fv, j, d, cls, total_eff):
    jf = j.astype(_f32)
    half = d // 2
    rem = d - half
    one = jnp.ones((L,), _f32)
    zero = jnp.zeros((L,), _f32)
    t1 = jf / jnp.maximum(half - 1, 1).astype(_f32)
    t2 = (jf - half.astype(_f32)) / jnp.maximum(rem - 1, 1).astype(_f32)
    lt = j < half
    ws_lin = jnp.where(lt, 1.0 - t1, zero)
    wm_lin = jnp.where(lt, t1, 1.0 - t2)
    we_lin = jnp.where(lt, zero, t2)
    j0 = j == 0
    jl = j == d - 1
    ws_pl = jnp.where(j0, one, zero)
    we_pl = jnp.where(jl & ~j0, one, zero)
    wm_pl = jnp.where(~j0 & ~jl, one, zero)
    ispl = cls == 0
    ws4 = jnp.where(ispl, ws_pl, ws_lin)
    wm4 = jnp.where(ispl, wm_pl, wm_lin)
    we4 = jnp.where(ispl, we_pl, we_lin)
    f1 = lambda bcond: jnp.where(bcond, one, zero)
    d1, d2, d3 = d == 1, d == 2, d == 3
    ws = jnp.where(d1, zero, jnp.where(d2, f1(j0), jnp.where(d3, f1(j0), ws4)))
    wm = jnp.where(d1, one, jnp.where(d2, zero, jnp.where(d3, f1(j == 1), wm4)))
    we = jnp.where(d1, zero, jnp.where(d2, f1(~j0), jnp.where(d3, f1(j >= 2), we4)))
    valid = fv < total_eff
    ws = jnp.where(valid, ws, zero)
    wm = jnp.where(valid, wm, zero)
    we = jnp.where(valid, we, zero)
    return ws, wm, we, valid


def _rows(shape=(BLK, FDIM), dt=_f32):
    return pltpu.VMEM(shape, dt)


@functools.partial(
    pl.kernel,
    mesh=_mesh,
    compiler_params=pltpu.CompilerParams(needs_layout_passes=False),
    out_type=(
        jax.ShapeDtypeStruct((B * MAXF, FDIM), _f32),
        jax.ShapeDtypeStruct((B * MAXF,), _i32),
    ),
    scratch_types=[
        pltpu.VMEM((N,), _i32),
        pltpu.VMEM((N,), _i32),
        pltpu.VMEM((N,), _i32),
        pltpu.VMEM((L,), _i32),
        pltpu.VMEM((FPT,), _i32),
        pltpu.VMEM((BLK,), _i32), pltpu.VMEM((BLK,), _f32),
        pltpu.VMEM((BLK,), _f32), pltpu.VMEM((BLK,), _f32),
        _rows(), _rows(), _rows(), _rows(),
        pltpu.VMEM((BLK,), _i32), pltpu.VMEM((BLK,), _f32),
        pltpu.VMEM((BLK,), _f32), pltpu.VMEM((BLK,), _f32),
        _rows(), _rows(), _rows(), _rows(),
        pltpu.SemaphoreType.DMA, pltpu.SemaphoreType.DMA,
        pltpu.SemaphoreType.DMA, pltpu.SemaphoreType.DMA,
        pltpu.SemaphoreType.DMA, pltpu.SemaphoreType.DMA,
        pltpu.SemaphoreType.DMA, pltpu.SemaphoreType.DMA,
    ],
)
def _render(s_hbm, m_hbm, e_hbm, dur_hbm, cls_hbm, mf_hbm,
            out_hbm, mask_hbm,
            dur_v, csum_v, cls_v, mf_v, mask_v,
            idx0, ws0, wm0, we0, s0, m0, e0, o0,
            idx1, ws1, wm1, we1, s1, m1, e1, o1,
            sem_s0, sem_m0, sem_e0, sem_o0,
            sem_s1, sem_m1, sem_e1, sem_o1):
    cid = lax.axis_index("c")
    sid = lax.axis_index("s")
    wid = sid * NC + cid
    bidx = wid // TPB
    q = wid % TPB
    f0 = q * FPT

    lane = lax.broadcasted_iota(_i32, (L,), 0)

    pltpu.sync_copy(dur_hbm.at[pl.ds(bidx * N, N)], dur_v)
    pltpu.sync_copy(cls_hbm.at[pl.ds(bidx * N, N)], cls_v)
    pltpu.sync_copy(mf_hbm, mf_v)

    def cs_body(i, carry_v):
        base = i * L
        dv = jnp.maximum(dur_v[pl.ds(base, L)], 0)
        csum_v[pl.ds(base, L)] = dv
        for sh in (1, 2, 4, 8):
            cur = csum_v[pl.ds(base, L)]
            prev = plsc.load_gather(csum_v, [base + jnp.maximum(lane - sh, 0)])
            csum_v[pl.ds(base, L)] = cur + jnp.where(lane >= sh, prev, 0)
        cv = csum_v[pl.ds(base, L)] + carry_v
        csum_v[pl.ds(base, L)] = cv
        return plsc.load_gather(csum_v, [jnp.zeros((L,), _i32) + (base + L - 1)])

    lax.fori_loop(0, N // L, cs_body, jnp.zeros((L,), _i32))

    total_v = plsc.load_gather(csum_v, [jnp.zeros((L,), _i32) + (N - 1)])
    mf_splat = mf_v[...]
    total_eff = jnp.minimum(total_v, mf_splat)

    def prefetch(g, idx_r, ws_r, wm_r, we_r, s_r, m_r, e_r, ss, sm, se):
        base = f0 + g * BLK
        for k in range(BLK // L):
            fv = base + k * L + lane
            pos = jnp.zeros((L,), _i32)
            bit = N // 2
            while bit >= 1:
                val = plsc.load_gather(csum_v, [pos + (bit - 1)])
                pos = pos + jnp.where(val <= fv, bit, 0)
                bit //= 2
            ph = pos
            offm = plsc.load_gather(csum_v, [jnp.maximum(ph - 1, 0)])
            off = jnp.where(ph >= 1, offm, 0)
            dsum = plsc.load_gather(csum_v, [ph])
            d = dsum - off
            j = fv - off
            cls = plsc.load_gather(cls_v, [ph])
            ws, wm, we, valid = _weights(fv, j, d, cls, total_eff)
            idx_r[pl.ds(k * L, L)] = bidx * N + ph
            ws_r[pl.ds(k * L, L)] = ws
            wm_r[pl.ds(k * L, L)] = wm
            we_r[pl.ds(k * L, L)] = we
            mask_v[pl.ds(g * BLK + k * L, L)] = valid.astype(_i32)
        pltpu.async_copy(s_hbm.at[idx_r], s_r, ss)
        pltpu.async_copy(m_hbm.at[idx_r], m_r, sm)
        pltpu.async_copy(e_hbm.at[idx_r], e_r, se)

    def wait_rows(idx_r, s_r, m_r, e_r, ss, sm, se):
        pltpu.make_async_copy(s_hbm.at[idx_r], s_r, ss).wait()
        pltpu.make_async_copy(m_hbm.at[idx_r], m_r, sm).wait()
        pltpu.make_async_copy(e_hbm.at[idx_r], e_r, se).wait()

    def compute(g, ws_r, wm_r, we_r, s_r, m_r, e_r, o_r, so):
        def row_body(r, carry):
            rsplat = jnp.zeros((L,), _i32) + r
            wsr = plsc.load_gather(ws_r, [rsplat])
            wmr = plsc.load_gather(wm_r, [rsplat])
            wer = plsc.load_gather(we_r, [rsplat])
            for kk in range(CHUNKS):
                sv = s_r[r, pl.ds(kk * L, L)]
                mv = m_r[r, pl.ds(kk * L, L)]
                ev = e_r[r, pl.ds(kk * L, L)]
                o_r[r, pl.ds(kk * L, L)] = wsr * sv + wmr * mv + wer * ev
            return carry

        lax.fori_loop(0, BLK, row_body, 0)
        pltpu.async_copy(
            o_r, out_hbm.at[pl.ds(bidx * MAXF + f0 + g * BLK, BLK)], so)

    def wait_out(g, o_r, so):
        pltpu.make_async_copy(
            o_r, out_hbm.at[pl.ds(bidx * MAXF + f0 + g * BLK, BLK)], so).wait()

    slot0 = (idx0, ws0, wm0, we0, s0, m0, e0, sem_s0, sem_m0, sem_e0)
    slot1 = (idx1, ws1, wm1, we1, s1, m1, e1, sem_s1, sem_m1, sem_e1)

    prefetch(jnp.asarray(0, _i32), *slot0)
    prefetch(jnp.asarray(1, _i32), *slot1)

    def pair_body(i, _):
        g0 = 2 * i
        g1 = 2 * i + 1
        wait_rows(idx0, s0, m0, e0, sem_s0, sem_m0, sem_e0)
        pl.when(i > 0)(lambda: wait_out(g0 - 2, o0, sem_o0))
        compute(g0, ws0, wm0, we0, s0, m0, e0, o0, sem_o0)
        prefetch(jnp.minimum(g0 + 2, NBLK - 1), *slot0)
        wait_rows(idx1, s1, m1, e1, sem_s1, sem_m1, sem_e1)
        pl.when(i > 0)(lambda: wait_out(g1 - 2, o1, sem_o1))
        compute(g1, ws1, wm1, we1, s1, m1, e1, o1, sem_o1)
        prefetch(jnp.minimum(g1 + 2, NBLK - 1), *slot1)
        return _

    lax.fori_loop(0, NBLK // 2, pair_body, 0)

    wait_rows(idx0, s0, m0, e0, sem_s0, sem_m0, sem_e0)
    wait_rows(idx1, s1, m1, e1, sem_s1, sem_m1, sem_e1)
    wait_out(NBLK - 2, o0, sem_o0)
    wait_out(NBLK - 1, o1, sem_o1)

    pltpu.sync_copy(mask_v, mask_hbm.at[pl.ds(bidx * MAXF + f0, FPT)])


def kernel(start, mid, end, durations, render_class, max_frames):
    s = start.reshape(B * N, FDIM)
    m = mid.reshape(B * N, FDIM)
    e = end.reshape(B * N, FDIM)
    dur = durations.reshape(B * N)
    cls = render_class.reshape(B * N)
    mf = jnp.full((L,), max_frames, _i32)
    out_flat, mask_flat = _render(s, m, e, dur, cls, mf)
    out = out_flat.reshape(B, MAXF, FDIM)
    mask = mask_flat.reshape(B, MAXF) != 0
    return out, mask
